# Initial kernel scaffold; baseline (speedup 1.0000x reference)
import jax
import jax.numpy as jnp
from jax.experimental import pallas as pl


def kernel(agent_idx, task_idx, theta, beta):
    raise NotImplementedError("write your pallas kernel here")


# speedup vs baseline: 1.3175x; 1.3175x over previous
import functools

import jax
import jax.numpy as jnp
from jax import lax
from jax.experimental import pallas as pl
from jax.experimental.pallas import tpu as pltpu
from jax.experimental.pallas import tpu_sc as plsc

BATCH = 16384
NUM_WORKERS = 32
CHUNK = 128
PER_WORKER = BATCH // NUM_WORKERS
NUM_CHUNKS = PER_WORKER // CHUNK
LANES = 16


def _irt_body(agent_r, task_r, theta_r, beta_r, out_r, idx_a, idx_t, th, be, sem):
    nc = plsc.get_sparse_core_info().num_cores
    wid = lax.axis_index("s") * nc + lax.axis_index("c")

    pltpu.sync_copy(agent_r.at[wid], idx_a)
    pltpu.sync_copy(task_r.at[wid], idx_t)

    copies = []
    for j in range(NUM_CHUNKS):
        copies.append(pltpu.async_copy(theta_r.at[idx_a.at[j]], th.at[j], sem))
        copies.append(pltpu.async_copy(beta_r.at[idx_t.at[j]], be.at[j], sem))
    for c in copies:
        c.wait()

    for j in range(NUM_CHUNKS):
        for i in range(CHUNK // LANES):
            s = pl.ds(i * LANES, LANES)
            th[j, s] = th[j, s] - be[j, s]

    pltpu.sync_copy(th, out_r.at[wid])


@jax.jit
def _irt(agent_idx, task_idx, theta, beta):
    mesh = plsc.VectorSubcoreMesh(core_axis_name="c", subcore_axis_name="s")
    run = pl.kernel(
        _irt_body,
        out_type=jax.ShapeDtypeStruct((NUM_WORKERS, NUM_CHUNKS, CHUNK), jnp.float32),
        mesh=mesh,
        scratch_types=[
            pltpu.VMEM((NUM_CHUNKS, CHUNK), jnp.int32),
            pltpu.VMEM((NUM_CHUNKS, CHUNK), jnp.int32),
            pltpu.VMEM((NUM_CHUNKS, CHUNK), jnp.float32),
            pltpu.VMEM((NUM_CHUNKS, CHUNK), jnp.float32),
            pltpu.SemaphoreType.DMA,
        ],
    )
    a = agent_idx.astype(jnp.int32).reshape(NUM_WORKERS, NUM_CHUNKS, CHUNK)
    t = task_idx.astype(jnp.int32).reshape(NUM_WORKERS, NUM_CHUNKS, CHUNK)
    out = run(a, t, theta.reshape(-1), beta.reshape(-1))
    return out.reshape(BATCH)


def kernel(agent_idx, task_idx, theta, beta):
    return _irt(agent_idx, task_idx, theta, beta)


# speedup vs baseline: 1.3436x; 1.0198x over previous
import functools

import jax
import jax.numpy as jnp
from jax import lax
from jax.experimental import pallas as pl
from jax.experimental.pallas import tpu as pltpu
from jax.experimental.pallas import tpu_sc as plsc

BATCH = 16384
NUM_WORKERS = 32
CHUNK = 128
PER_WORKER = BATCH // NUM_WORKERS
NUM_CHUNKS = PER_WORKER // CHUNK
LANES = 16


def _irt_body(agent_r, task_r, theta_r, beta_r, out_r, idx_a, idx_t, th, be, sem):
    nc = plsc.get_sparse_core_info().num_cores
    wid = lax.axis_index("s") * nc + lax.axis_index("c")

    ca = pltpu.async_copy(agent_r.at[wid], idx_a, sem)
    ct = pltpu.async_copy(task_r.at[wid], idx_t, sem)
    ca.wait()
    ct.wait()

    copies = []
    for j in range(NUM_CHUNKS):
        copies.append(pltpu.async_copy(theta_r.at[idx_a.at[j]], th.at[j], sem))
        copies.append(pltpu.async_copy(beta_r.at[idx_t.at[j]], be.at[j], sem))
    for c in copies:
        c.wait()

    for j in range(NUM_CHUNKS):
        for i in range(CHUNK // LANES):
            s = pl.ds(i * LANES, LANES)
            th[j, s] = th[j, s] - be[j, s]

    pltpu.sync_copy(th, out_r.at[wid])


@jax.jit
def _irt(agent_idx, task_idx, theta, beta):
    mesh = plsc.VectorSubcoreMesh(core_axis_name="c", subcore_axis_name="s")
    run = pl.kernel(
        _irt_body,
        out_type=jax.ShapeDtypeStruct((NUM_WORKERS, NUM_CHUNKS, CHUNK), jnp.float32),
        mesh=mesh,
        scratch_types=[
            pltpu.VMEM((NUM_CHUNKS, CHUNK), jnp.int32),
            pltpu.VMEM((NUM_CHUNKS, CHUNK), jnp.int32),
            pltpu.VMEM((NUM_CHUNKS, CHUNK), jnp.float32),
            pltpu.VMEM((NUM_CHUNKS, CHUNK), jnp.float32),
            pltpu.SemaphoreType.DMA,
        ],
    )
    a = agent_idx.astype(jnp.int32).reshape(NUM_WORKERS, NUM_CHUNKS, CHUNK)
    t = task_idx.astype(jnp.int32).reshape(NUM_WORKERS, NUM_CHUNKS, CHUNK)
    out = run(a, t, theta.reshape(-1), beta.reshape(-1))
    return out.reshape(BATCH)


def kernel(agent_idx, task_idx, theta, beta):
    return _irt(agent_idx, task_idx, theta, beta)


# speedup vs baseline: 1.4719x; 1.0955x over previous
import jax
import jax.numpy as jnp
from jax import lax
from jax.experimental import pallas as pl
from jax.experimental.pallas import tpu as pltpu
from jax.experimental.pallas import tpu_sc as plsc

BATCH = 16384
NUM_WORKERS = 32
NUM_SUBCORES = 16
CHUNK = 128
PER_WORKER = BATCH // NUM_WORKERS
NUM_CHUNKS = PER_WORKER // CHUNK
LANES = 16
TABLE = 100000


def _irt_body(agent_r, task_r, theta_r, beta_r, out_r,
              idx_a, idx_t, th, be, tab_t, tab_b, sem):
    nc = plsc.get_sparse_core_info().num_cores
    sid = lax.axis_index("s")
    wid = sid * nc + lax.axis_index("c")

    ca = pltpu.async_copy(agent_r.at[wid], idx_a, sem)
    ct = pltpu.async_copy(task_r.at[wid], idx_t, sem)

    @pl.when(sid == 0)
    def _():
        pltpu.sync_copy(theta_r.at[0], tab_t)

    @pl.when(sid == 1)
    def _():
        pltpu.sync_copy(beta_r.at[0], tab_b)

    plsc.subcore_barrier()
    ca.wait()
    ct.wait()

    copies = []
    for j in range(NUM_CHUNKS):
        copies.append(pltpu.async_copy(tab_t.at[idx_a.at[j]], th.at[j], sem))
        copies.append(pltpu.async_copy(tab_b.at[idx_t.at[j]], be.at[j], sem))
    for c in copies:
        c.wait()

    for j in range(NUM_CHUNKS):
        for i in range(CHUNK // LANES):
            s = pl.ds(i * LANES, LANES)
            th[j, s] = th[j, s] - be[j, s]

    pltpu.sync_copy(th, out_r.at[wid])


@jax.jit
def _irt(agent_idx, task_idx, theta, beta):
    mesh = plsc.VectorSubcoreMesh(core_axis_name="c", subcore_axis_name="s")
    run = pl.kernel(
        _irt_body,
        out_type=jax.ShapeDtypeStruct((NUM_WORKERS, NUM_CHUNKS, CHUNK), jnp.float32),
        mesh=mesh,
        scratch_types=[
            pltpu.VMEM((NUM_CHUNKS, CHUNK), jnp.int32),
            pltpu.VMEM((NUM_CHUNKS, CHUNK), jnp.int32),
            pltpu.VMEM((NUM_CHUNKS, CHUNK), jnp.float32),
            pltpu.VMEM((NUM_CHUNKS, CHUNK), jnp.float32),
            pltpu.VMEM_SHARED((TABLE,), jnp.float32),
            pltpu.VMEM_SHARED((TABLE,), jnp.float32),
            pltpu.SemaphoreType.DMA,
        ],
    )
    a = agent_idx.astype(jnp.int32).reshape(NUM_WORKERS, NUM_CHUNKS, CHUNK)
    t = task_idx.astype(jnp.int32).reshape(NUM_WORKERS, NUM_CHUNKS, CHUNK)
    out = run(a, t, theta.reshape(1, TABLE), beta.reshape(1, TABLE))
    return out.reshape(BATCH)


def kernel(agent_idx, task_idx, theta, beta):
    return _irt(agent_idx, task_idx, theta, beta)


# speedup vs baseline: 1.4734x; 1.0010x over previous
import jax
import jax.numpy as jnp
from jax import lax
from jax.experimental import pallas as pl
from jax.experimental.pallas import tpu as pltpu
from jax.experimental.pallas import tpu_sc as plsc

BATCH = 16384
NUM_WORKERS = 32
NUM_SUBCORES = 16
CHUNK = 128
PER_WORKER = BATCH // NUM_WORKERS
NUM_CHUNKS = PER_WORKER // CHUNK
LANES = 16
TABLE = 100000
BULK = 6144
TAIL_OFF = BULK * NUM_SUBCORES
TAIL = TABLE - TAIL_OFF
TAIL_PAD = 2048


def _irt_body(agent_r, task_r, theta_r, beta_r, tail_t_r, tail_b_r, out_r,
              idx_a, idx_t, th, be, tab_t, tab_b, sem):
    nc = plsc.get_sparse_core_info().num_cores
    sid = lax.axis_index("s")
    wid = sid * nc + lax.axis_index("c")

    ca = pltpu.async_copy(agent_r.at[wid], idx_a, sem)
    ct = pltpu.async_copy(task_r.at[wid], idx_t, sem)

    base = pl.multiple_of(sid * BULK, 128)
    cs = [pltpu.async_copy(theta_r.at[0, pl.ds(base, BULK)],
                           tab_t.at[pl.ds(base, BULK)], sem),
          pltpu.async_copy(beta_r.at[0, pl.ds(base, BULK)],
                           tab_b.at[pl.ds(base, BULK)], sem)]

    @pl.when(sid == 0)
    def _():
        pltpu.sync_copy(tail_t_r, tab_t.at[pl.ds(TAIL_OFF, TAIL_PAD)])

    @pl.when(sid == 1)
    def _():
        pltpu.sync_copy(tail_b_r, tab_b.at[pl.ds(TAIL_OFF, TAIL_PAD)])

    for c in cs:
        c.wait()
    plsc.subcore_barrier()
    ca.wait()
    ct.wait()

    copies = []
    for j in range(NUM_CHUNKS):
        copies.append(pltpu.async_copy(tab_t.at[idx_a.at[j]], th.at[j], sem))
        copies.append(pltpu.async_copy(tab_b.at[idx_t.at[j]], be.at[j], sem))
    for c in copies:
        c.wait()

    for j in range(NUM_CHUNKS):
        for i in range(CHUNK // LANES):
            s = pl.ds(i * LANES, LANES)
            th[j, s] = th[j, s] - be[j, s]

    pltpu.sync_copy(th, out_r.at[wid])


@jax.jit
def _irt(agent_idx, task_idx, theta, beta):
    mesh = plsc.VectorSubcoreMesh(core_axis_name="c", subcore_axis_name="s")
    run = pl.kernel(
        _irt_body,
        out_type=jax.ShapeDtypeStruct((NUM_WORKERS, NUM_CHUNKS, CHUNK), jnp.float32),
        mesh=mesh,
        scratch_types=[
            pltpu.VMEM((NUM_CHUNKS, CHUNK), jnp.int32),
            pltpu.VMEM((NUM_CHUNKS, CHUNK), jnp.int32),
            pltpu.VMEM((NUM_CHUNKS, CHUNK), jnp.float32),
            pltpu.VMEM((NUM_CHUNKS, CHUNK), jnp.float32),
            pltpu.VMEM_SHARED((TAIL_OFF + TAIL_PAD,), jnp.float32),
            pltpu.VMEM_SHARED((TAIL_OFF + TAIL_PAD,), jnp.float32),
            pltpu.SemaphoreType.DMA,
        ],
    )
    a = agent_idx.astype(jnp.int32).reshape(NUM_WORKERS, NUM_CHUNKS, CHUNK)
    t = task_idx.astype(jnp.int32).reshape(NUM_WORKERS, NUM_CHUNKS, CHUNK)
    out = run(a, t, theta.reshape(1, TABLE), beta.reshape(1, TABLE),
              jnp.pad(theta[TAIL_OFF:, 0], (0, TAIL_PAD - TAIL)),
              jnp.pad(beta[TAIL_OFF:, 0], (0, TAIL_PAD - TAIL)))
    return out.reshape(BATCH)


def kernel(agent_idx, task_idx, theta, beta):
    return _irt(agent_idx, task_idx, theta, beta)
